# Initial kernel scaffold; baseline (speedup 1.0000x reference)
import jax
import jax.numpy as jnp
from jax.experimental import pallas as pl


def kernel(x, edge_index, edge_attr, batch, W1, b1, W2, b2, W3, b3, W4, b4, conv_w, conv_b, gamma, beta):
    raise NotImplementedError("write your pallas kernel here")


# speedup vs baseline: 5.3523x; 5.3523x over previous
import functools

import jax
import jax.numpy as jnp
from jax import lax
from jax.experimental import pallas as pl
from jax.experimental.pallas import tpu as pltpu
from jax.experimental.pallas import tpu_sc as plsc

N = 21120
E = 443520
BS = 4
T = 240
G = BS * T
IC = 14
D1, D2, D3, D4 = 16, 32, 64, 152
DH = 80
D4A = 76
DCAP = 16
NCLS = 17
NCH = DCAP * NCLS
EPS = 1e-3

NCORE = 2
NSUB = 16
K = 128
EP = 32 * K * 109
CPT_SPLIT = EP // 32 // K
CPT_FULL = EP // 16 // K
RPT = N // NSUB


def _sc_scatter(D, full_edges, with_gather):
    mesh = plsc.VectorSubcoreMesh(core_axis_name="c", subcore_axis_name="s")
    nchunks = CPT_FULL if full_edges else CPT_SPLIT

    def body(src_hbm, dst_hbm, wexp_hbm, y_hbm, zeros_hbm, out_hbm,
             srcv, dstv, rows, wv, acc, sem):
        c = lax.axis_index("c")
        s = lax.axis_index("s")
        pltpu.sync_copy(zeros_hbm.at[pl.ds(s * RPT, RPT)],
                        acc.at[pl.ds(s * RPT, RPT)])
        plsc.subcore_barrier()

        if full_edges:
            ebase = s * (EP // NSUB)
        else:
            ebase = (c * NSUB + s) * (EP // 32)

        def chunk(k, carry):
            b_e = ebase + k * K
            pltpu.sync_copy(dst_hbm.at[pl.ds(b_e, K)], dstv)
            pltpu.sync_copy(wexp_hbm.at[pl.ds(b_e, K)], wv)
            if with_gather:
                if full_edges:
                    pltpu.sync_copy(src_hbm.at[pl.ds(c * EP + b_e, K)], srcv)
                else:
                    pltpu.sync_copy(src_hbm.at[pl.ds(b_e, K)], srcv)
                pltpu.async_copy(y_hbm.at[srcv], rows, sem).wait()

                def rowbody(r, cc):
                    wvec = wv[r, :]
                    for j in range(D // 16):
                        sl = pl.ds(16 * j, 16)
                        rows[r, sl] = rows[r, sl] * wvec
                    return cc

                lax.fori_loop(0, K, rowbody, 0)
                pltpu.sync_copy(rows, acc.at[dstv], add=True)
            else:
                pltpu.sync_copy(wv, acc.at[dstv], add=True)
            return carry

        lax.fori_loop(0, nchunks, chunk, 0)
        plsc.subcore_barrier()
        pltpu.sync_copy(acc.at[pl.ds(s * RPT, RPT)],
                        out_hbm.at[pl.ds(c * N + s * RPT, RPT)])

    return pl.kernel(
        body,
        out_type=jax.ShapeDtypeStruct((2 * N, D), jnp.float32),
        mesh=mesh,
        scratch_types=[
            pltpu.VMEM((K,), jnp.int32),
            pltpu.VMEM((K,), jnp.int32),
            pltpu.VMEM((K, D), jnp.float32),
            pltpu.VMEM((K, 16), jnp.float32),
            pltpu.VMEM_SHARED((N, D), jnp.float32),
            pltpu.SemaphoreType.DMA,
        ],
        compiler_params=pltpu.CompilerParams(use_tc_tiling_on_sc=False),
    )


_NB = 16


def _row(d):
    return pl.BlockSpec((RPT, d), lambda i: (i, 0))


def _const(r, d):
    return pl.BlockSpec((r, d), lambda i: (0, 0))


def _tc_prep(degA, degB, x, W1):
    def body(degA_ref, degB_ref, x_ref, w_ref, dis_ref, y_ref):
        deg = degA_ref[:, 0:1] + degB_ref[:, 0:1] + 1.0
        dis = jnp.where(deg > 0, lax.rsqrt(jnp.maximum(deg, 1e-12)), 0.0)
        dis_ref[...] = dis
        y_ref[...] = dis * jnp.dot(x_ref[...], w_ref[...],
                                   preferred_element_type=jnp.float32)
    return pl.pallas_call(
        body,
        grid=(_NB,),
        in_specs=[_row(D1), _row(D1), _row(IC), _const(IC, D1)],
        out_specs=(_row(1), _row(D1)),
        out_shape=(jax.ShapeDtypeStruct((N, 1), jnp.float32),
                   jax.ShapeDtypeStruct((N, D1), jnp.float32)),
    )(degA, degB, x, W1)


def _tc_mid(accA, accB, y, dis, b, Wn):
    Dk = Wn.shape[0]
    Dn = Wn.shape[1]
    def body(accA_ref, accB_ref, y_ref, dis_ref, b_ref, w_ref, yn_ref):
        dis = dis_ref[...]
        h = jnp.maximum(
            dis * (accA_ref[...] + accB_ref[...] + y_ref[...]) + b_ref[...], 0.0)
        yn_ref[...] = dis * jnp.dot(h, w_ref[...],
                                    preferred_element_type=jnp.float32)
    return pl.pallas_call(
        body,
        grid=(_NB,),
        in_specs=[_row(Dk), _row(Dk), _row(Dk), _row(1),
                  _const(1, Dk), _const(Dk, Dn)],
        out_specs=_row(Dn),
        out_shape=jax.ShapeDtypeStruct((N, Dn), jnp.float32),
    )(accA, accB, y, dis, b.reshape(1, -1), Wn)


def _tc_mid4(accA, accB, y, dis, b, W4a, W4b):
    def body(accA_ref, accB_ref, y_ref, dis_ref, b_ref, wa_ref, wb_ref,
             ya_ref, yb_ref):
        dis = dis_ref[...]
        h = jnp.maximum(
            dis * (accA_ref[...] + accB_ref[...] + y_ref[...]) + b_ref[...], 0.0)
        ya_ref[...] = dis * jnp.dot(h, wa_ref[...],
                                    preferred_element_type=jnp.float32)
        yb_ref[...] = dis * jnp.dot(h, wb_ref[...],
                                    preferred_element_type=jnp.float32)
    return pl.pallas_call(
        body,
        grid=(_NB,),
        in_specs=[_row(D3), _row(D3), _row(D3), _row(1),
                  _const(1, D3), _const(D3, DH), _const(D3, DH)],
        out_specs=(_row(DH), _row(DH)),
        out_shape=(jax.ShapeDtypeStruct((N, DH), jnp.float32),
                   jax.ShapeDtypeStruct((N, DH), jnp.float32)),
    )(accA, accB, y, dis, b.reshape(1, -1), W4a, W4b)


def _tc_head(acc4a, acc4b, y4a, y4b, dis, batch3, b4a, b4b,
             w0a, w0b, w1a, w1b, w2a, w2b, convb, gcol, bcol, Rm):
    NB = 16

    def body(acc4a_ref, acc4b_ref, y4a_ref, y4b_ref, dis_ref, batch_ref,
             b4a_ref, b4b_ref, w0a_ref, w0b_ref, w1a_ref, w1b_ref,
             w2a_ref, w2b_ref, convb_ref, gcol_ref, bcol_ref, rm_ref,
             out_ref, pa, pb, cnt):
        i = pl.program_id(0)

        @pl.when(i == 0)
        def _init():
            pa[...] = jnp.zeros_like(pa)
            pb[...] = jnp.zeros_like(pb)
            cnt[...] = jnp.zeros_like(cnt)

        dis = dis_ref[...]
        h4a = jnp.maximum(dis * (acc4a_ref[...] + y4a_ref[...]) + b4a_ref[...], 0.0)
        h4b = jnp.maximum(dis * (acc4b_ref[...] + y4b_ref[...]) + b4b_ref[...], 0.0)
        bvals = batch_ref[0]
        iot = lax.broadcasted_iota(jnp.int32, (G, RPT), 0)
        mask = (bvals == iot).astype(jnp.float32)
        dn = (((1,), (0,)), ((), ()))
        pa[...] += lax.dot_general(mask, h4a, dn, preferred_element_type=jnp.float32)
        pb[...] += lax.dot_general(mask, h4b, dn, preferred_element_type=jnp.float32)
        cnt[...] += jnp.sum(mask, axis=1, keepdims=True)

        @pl.when(i == NB - 1)
        def _epilogue():
            cntc = jnp.maximum(cnt[...], 1.0)
            Pa = pa[...] / cntc
            Pb = pb[...] / cntc
            r = lax.broadcasted_iota(jnp.int32, (G, 1), 0)
            mprev = (r % T != 0).astype(jnp.float32)
            mnext = (r % T != T - 1).astype(jnp.float32)
            z = jnp.zeros((1, DH), jnp.float32)
            Pa_p = jnp.concatenate([z, Pa[:-1]], axis=0) * mprev
            Pb_p = jnp.concatenate([z, Pb[:-1]], axis=0) * mprev
            Pa_n = jnp.concatenate([Pa[1:], z], axis=0) * mnext
            Pb_n = jnp.concatenate([Pb[1:], z], axis=0) * mnext

            def mm(a, w_ref):
                return jnp.dot(a, w_ref[...], preferred_element_type=jnp.float32)

            C = (mm(Pa_p, w0a_ref) + mm(Pb_p, w0b_ref)
                 + mm(Pa, w1a_ref) + mm(Pb, w1b_ref)
                 + mm(Pa_n, w2a_ref) + mm(Pb_n, w2b_ref)) + convb_ref[...]
            s1 = jnp.sum(C, axis=1, keepdims=True)
            s2 = jnp.sum(C * C, axis=1, keepdims=True)
            S1 = s1[0:T] + s1[T:2 * T] + s1[2 * T:3 * T] + s1[3 * T:]
            S2 = s2[0:T] + s2[T:2 * T] + s2[2 * T:3 * T] + s2[3 * T:]
            cnt_bn = float(BS * NCH)
            mu = S1 / cnt_bn
            var = S2 / cnt_bn - mu * mu
            muf = jnp.concatenate([mu, mu, mu, mu], axis=0)
            varf = jnp.concatenate([var, var, var, var], axis=0)
            bn = (C - muf) * lax.rsqrt(varf + EPS) * gcol_ref[...] + bcol_ref[...]
            sg = jax.nn.sigmoid(bn)
            q = (sg - 0.5) ** 2
            o = jnp.sqrt(jnp.dot(q, rm_ref[...],
                                 preferred_element_type=jnp.float32) * (4.0 / DCAP))
            out_ref[...] = o

    row_spec = pl.BlockSpec((RPT, DH), lambda i: (i, 0))
    return pl.pallas_call(
        body,
        grid=(NB,),
        in_specs=[
            row_spec, row_spec, row_spec, row_spec,
            pl.BlockSpec((RPT, 1), lambda i: (i, 0)),
            pl.BlockSpec((1, 1, RPT), lambda i: (i, 0, 0)),
            pl.BlockSpec((1, DH), lambda i: (0, 0)),
            pl.BlockSpec((1, DH), lambda i: (0, 0)),
            pl.BlockSpec((DH, NCH), lambda i: (0, 0)),
            pl.BlockSpec((DH, NCH), lambda i: (0, 0)),
            pl.BlockSpec((DH, NCH), lambda i: (0, 0)),
            pl.BlockSpec((DH, NCH), lambda i: (0, 0)),
            pl.BlockSpec((DH, NCH), lambda i: (0, 0)),
            pl.BlockSpec((DH, NCH), lambda i: (0, 0)),
            pl.BlockSpec((1, NCH), lambda i: (0, 0)),
            pl.BlockSpec((G, 1), lambda i: (0, 0)),
            pl.BlockSpec((G, 1), lambda i: (0, 0)),
            pl.BlockSpec((NCH, NCLS), lambda i: (0, 0)),
        ],
        out_specs=pl.BlockSpec((G, NCLS), lambda i: (0, 0)),
        out_shape=jax.ShapeDtypeStruct((G, NCLS), jnp.float32),
        scratch_shapes=[
            pltpu.VMEM((G, DH), jnp.float32),
            pltpu.VMEM((G, DH), jnp.float32),
            pltpu.VMEM((G, 1), jnp.float32),
        ],
    )(acc4a, acc4b, y4a, y4b, dis, batch3, b4a, b4b,
      w0a, w0b, w1a, w1b, w2a, w2b, convb, gcol, bcol, Rm)


def kernel(x, edge_index, edge_attr, batch, W1, b1, W2, b2, W3, b3, W4, b4,
           conv_w, conv_b, gamma, beta):
    f32 = jnp.float32
    src = edge_index[0].astype(jnp.int32)
    dst = edge_index[1].astype(jnp.int32)
    w = edge_attr[:, 4].astype(f32)

    pad = EP - E
    srcp = jnp.concatenate([src, jnp.zeros((pad,), jnp.int32)])
    dstp = jnp.concatenate([dst, jnp.zeros((pad,), jnp.int32)])
    wexp = jnp.broadcast_to(
        jnp.concatenate([w, jnp.zeros((pad,), f32)])[:, None], (EP, 16))
    src4 = jnp.concatenate([srcp, srcp + N])

    z16 = jnp.zeros((N, D1), f32)
    z32 = jnp.zeros((N, D2), f32)
    z64 = jnp.zeros((N, D3), f32)
    z80 = jnp.zeros((N, DH), f32)

    W4a = jnp.pad(W4[:, :D4A], ((0, 0), (0, DH - D4A)))
    W4b = jnp.pad(W4[:, D4A:], ((0, 0), (0, DH - (D4 - D4A))))
    b4a = jnp.pad(b4[:D4A], (0, DH - D4A)).reshape(1, DH)
    b4b = jnp.pad(b4[D4A:], (0, DH - (D4 - D4A))).reshape(1, DH)

    wk = conv_w[:, :, :, 0]
    def _split(k):
        m = wk[:, :, k].T
        return (jnp.pad(m[:D4A], ((0, DH - D4A), (0, 0))),
                jnp.pad(m[D4A:], ((0, DH - (D4 - D4A)), (0, 0))))
    w0a, w0b = _split(0)
    w1a, w1b = _split(1)
    w2a, w2b = _split(2)
    convb = conv_b.reshape(1, NCH)
    gcol = jnp.concatenate([gamma] * BS).reshape(G, 1)
    bcol = jnp.concatenate([beta] * BS).reshape(G, 1)
    cidx = jnp.arange(NCH, dtype=jnp.int32)
    Rm = (cidx[:, None] % NCLS ==
          jnp.arange(NCLS, dtype=jnp.int32)[None, :]).astype(f32)
    batch3 = batch.astype(jnp.int32).reshape(16, 1, RPT)

    deg2 = _sc_scatter(D1, False, False)(srcp, dstp, wexp, z16, z16)
    dis, y1 = _tc_prep(deg2[:N], deg2[N:], x.astype(f32), W1)

    acc1 = _sc_scatter(D1, False, True)(srcp, dstp, wexp, y1, z16)
    y2 = _tc_mid(acc1[:N], acc1[N:], y1, dis, b1, W2)
    acc2 = _sc_scatter(D2, False, True)(srcp, dstp, wexp, y2, z32)
    y3 = _tc_mid(acc2[:N], acc2[N:], y2, dis, b2, W3)
    acc3 = _sc_scatter(D3, False, True)(srcp, dstp, wexp, y3, z64)
    y4a, y4b = _tc_mid4(acc3[:N], acc3[N:], y3, dis, b3, W4a, W4b)

    y4 = jnp.concatenate([y4a, y4b], axis=0)
    acc4 = _sc_scatter(DH, True, True)(src4, dstp, wexp, y4, z80)

    out = _tc_head(acc4[:N], acc4[N:], y4a, y4b, dis, batch3, b4a, b4b,
                   w0a, w0b, w1a, w1b, w2a, w2b, convb, gcol, bcol, Rm)
    return out.reshape(BS, T, NCLS)


# speedup vs baseline: 9.5961x; 1.7929x over previous
import jax
import jax.numpy as jnp
from jax import lax
from jax.experimental import pallas as pl
from jax.experimental.pallas import tpu as pltpu
from jax.experimental.pallas import tpu_sc as plsc

N = 21120
E = 443520
BS = 4
T = 240
G = BS * T
IC = 14
D1, D2, D3, D4 = 16, 32, 64, 152
DCAP = 16
NCLS = 17
NCH = DCAP * NCLS
EPS = 1e-3

NCORE = 2
NSUB = 16
K = 128
BLK = 16
EP = 32 * K * 112
ECH = EP // K
CPT = EP // 32 // K
NBUF = 3
RPT = N // NSUB


def _sc_scatter(D, with_gather):
    mesh = plsc.VectorSubcoreMesh(core_axis_name="c", subcore_axis_name="s")

    def body(src_hbm, dst_hbm, w_hbm, y_hbm, zeros_hbm, out_hbm,
             srcv, dstv, rows, wv, wrep, acc, gsem, ssem):
        c = lax.axis_index("c")
        s = lax.axis_index("s")
        pltpu.sync_copy(zeros_hbm.at[pl.ds(s * RPT, RPT)],
                        acc.at[pl.ds(s * RPT, RPT)])
        plsc.subcore_barrier()

        cbase = (c * NSUB + s) * CPT

        def blk_body(bi, carry):
            crow = cbase + bi * BLK
            pltpu.sync_copy(dst_hbm.at[pl.ds(crow, BLK)], dstv)
            if with_gather:
                pltpu.sync_copy(w_hbm.at[pl.ds(crow * K, BLK * K)], wv)
                pltpu.sync_copy(src_hbm.at[pl.ds(crow, BLK)], srcv)
                sd = [None] * BLK
                cp = pltpu.async_copy(y_hbm.at[srcv.at[0]], rows.at[0], gsem)
                for j in range(BLK):
                    if j + 1 < BLK:
                        if j - 2 >= 0:
                            sd[j - 2].wait()
                        nxt = pltpu.async_copy(y_hbm.at[srcv.at[j + 1]],
                                               rows.at[(j + 1) % NBUF], gsem)
                    cp.wait()
                    buf = j % NBUF

                    @plsc.parallel_loop(0, K // 16, step=1)
                    def _rows(g):
                        wg = wv[pl.ds(j * K + 16 * g, 16)]
                        for rr in range(16):
                            wvec = jnp.full((16,), wg[rr], jnp.float32)
                            r = 16 * g + rr
                            for jj in range(D // 16):
                                sl = pl.ds(16 * jj, 16)
                                rows[buf, r, sl] = rows[buf, r, sl] * wvec

                    sd[j] = pltpu.async_copy(rows.at[buf], acc.at[dstv.at[j]],
                                             ssem, add=True)
                    if j + 1 < BLK:
                        cp = nxt
                for jd in range(max(0, BLK - NBUF), BLK):
                    sd[jd].wait()
            else:
                pltpu.sync_copy(w_hbm.at[pl.ds(crow * K, BLK * K)], wrep)
                for j in range(BLK):
                    pltpu.sync_copy(wrep.at[pl.ds(j * K, K)],
                                    acc.at[dstv.at[j]], add=True)
            return carry

        lax.fori_loop(0, CPT // BLK, blk_body, 0)
        plsc.subcore_barrier()
        pltpu.sync_copy(acc.at[pl.ds(s * RPT, RPT)],
                        out_hbm.at[pl.ds(c * N + s * RPT, RPT)])

    if with_gather:
        data_scratch = [
            pltpu.VMEM((NBUF, K, D), jnp.float32),
            pltpu.VMEM((BLK * K,), jnp.float32),
        ]
    else:
        data_scratch = [
            pltpu.VMEM((NBUF, K, D), jnp.float32),
            pltpu.VMEM((BLK * K,), jnp.float32),
            pltpu.VMEM((BLK * K, 16), jnp.float32),
        ]

    def wrapped(src_hbm, dst_hbm, w_hbm, y_hbm, zeros_hbm, out_hbm,
                srcv, dstv, *rest):
        if with_gather:
            rows, wv, acc, gsem, ssem = rest
            wrep = None
        else:
            rows, wv, wrep, acc, gsem, ssem = rest
        return body(src_hbm, dst_hbm, w_hbm, y_hbm, zeros_hbm, out_hbm,
                    srcv, dstv, rows, wv, wrep, acc, gsem, ssem)

    return pl.kernel(
        wrapped,
        out_type=jax.ShapeDtypeStruct((2 * N, D), jnp.float32),
        mesh=mesh,
        scratch_types=[
            pltpu.VMEM((BLK, K), jnp.int32),
            pltpu.VMEM((BLK, K), jnp.int32),
        ] + data_scratch + [
            pltpu.VMEM_SHARED((N, D), jnp.float32),
            pltpu.SemaphoreType.DMA,
            pltpu.SemaphoreType.DMA,
        ],
        compiler_params=pltpu.CompilerParams(use_tc_tiling_on_sc=False),
    )


_NB = 16


def _row(d):
    return pl.BlockSpec((RPT, d), lambda i: (i, 0))


def _rowB(d):
    return pl.BlockSpec((RPT, d), lambda i: (i + _NB, 0))


def _const(r, d):
    return pl.BlockSpec((r, d), lambda i: (0, 0))


def _tc_prep(deg2, xpad):
    def body(degA_ref, degB_ref, x_ref, dis_ref, y_ref):
        deg = degA_ref[:, 0:1] + degB_ref[:, 0:1] + 1.0
        dis = jnp.where(deg > 0, lax.rsqrt(jnp.maximum(deg, 1e-12)), 0.0)
        dis_ref[...] = dis
        y_ref[...] = dis * x_ref[...]
    return pl.pallas_call(
        body,
        grid=(_NB,),
        in_specs=[_row(D1), _rowB(D1), _row(D1)],
        out_specs=(_row(1), _row(D1)),
        out_shape=(jax.ShapeDtypeStruct((N, 1), jnp.float32),
                   jax.ShapeDtypeStruct((N, D1), jnp.float32)),
    )(deg2, deg2, xpad)


def _tc_mid(z2, y, dis, b, Wn):
    Dk = Wn.shape[0]
    Dn = Wn.shape[1]
    def body(zA_ref, zB_ref, y_ref, dis_ref, b_ref, w_ref, yn_ref):
        dis = dis_ref[...]
        t = zA_ref[...] + zB_ref[...] + y_ref[...]
        h = jnp.maximum(
            dis * jnp.dot(t, w_ref[...], preferred_element_type=jnp.float32)
            + b_ref[...], 0.0)
        yn_ref[...] = dis * h
    return pl.pallas_call(
        body,
        grid=(_NB,),
        in_specs=[_row(Dk), _rowB(Dk), _row(Dk), _row(1),
                  _const(1, Dn), _const(Dk, Dn)],
        out_specs=_row(Dn),
        out_shape=jax.ShapeDtypeStruct((N, Dn), jnp.float32),
    )(z2, z2, y, dis, b.reshape(1, -1), Wn)


def _tc_head(z4, y4, dis, batch3, b4, W4,
             w0, w1, w2, convb, gcol, bcol, Rm):
    def body(z4A_ref, z4B_ref, y4_ref, dis_ref, batch_ref, b4_ref, w4_ref,
             w0_ref, w1_ref, w2_ref, convb_ref, gcol_ref, bcol_ref, rm_ref,
             out_ref, pacc, cnt):
        i = pl.program_id(0)

        @pl.when(i == 0)
        def _init():
            pacc[...] = jnp.zeros_like(pacc)
            cnt[...] = jnp.zeros_like(cnt)

        dis = dis_ref[...]
        t = z4A_ref[...] + z4B_ref[...] + y4_ref[...]
        h4 = jnp.maximum(
            dis * jnp.dot(t, w4_ref[...], preferred_element_type=jnp.float32)
            + b4_ref[...], 0.0)
        bvals = batch_ref[0]
        iot = lax.broadcasted_iota(jnp.int32, (G, RPT), 0)
        mask = (bvals == iot).astype(jnp.float32)
        dn = (((1,), (0,)), ((), ()))
        pacc[...] += lax.dot_general(mask, h4, dn,
                                     preferred_element_type=jnp.float32)
        cnt[...] += jnp.sum(mask, axis=1, keepdims=True)

        @pl.when(i == _NB - 1)
        def _epilogue():
            cntc = jnp.maximum(cnt[...], 1.0)
            P = pacc[...] / cntc
            r = lax.broadcasted_iota(jnp.int32, (G, 1), 0)
            mprev = (r % T != 0).astype(jnp.float32)
            mnext = (r % T != T - 1).astype(jnp.float32)
            z = jnp.zeros((1, D4), jnp.float32)
            Pp = jnp.concatenate([z, P[:-1]], axis=0) * mprev
            Pn = jnp.concatenate([P[1:], z], axis=0) * mnext

            def mm(a, w_ref):
                return jnp.dot(a, w_ref[...], preferred_element_type=jnp.float32)

            C = mm(Pp, w0_ref) + mm(P, w1_ref) + mm(Pn, w2_ref) + convb_ref[...]
            s1 = jnp.sum(C, axis=1, keepdims=True)
            s2 = jnp.sum(C * C, axis=1, keepdims=True)
            S1 = s1[0:T] + s1[T:2 * T] + s1[2 * T:3 * T] + s1[3 * T:]
            S2 = s2[0:T] + s2[T:2 * T] + s2[2 * T:3 * T] + s2[3 * T:]
            cnt_bn = float(BS * NCH)
            mu = S1 / cnt_bn
            var = S2 / cnt_bn - mu * mu
            muf = jnp.concatenate([mu, mu, mu, mu], axis=0)
            varf = jnp.concatenate([var, var, var, var], axis=0)
            bn = (C - muf) * lax.rsqrt(varf + EPS) * gcol_ref[...] + bcol_ref[...]
            sg = jax.nn.sigmoid(bn)
            q = (sg - 0.5) ** 2
            o = jnp.sqrt(jnp.dot(q, rm_ref[...],
                                 preferred_element_type=jnp.float32) * (4.0 / DCAP))
            out_ref[...] = o

    return pl.pallas_call(
        body,
        grid=(_NB,),
        in_specs=[
            _row(D3), _rowB(D3), _row(D3), _row(1),
            pl.BlockSpec((1, 1, RPT), lambda i: (i, 0, 0)),
            _const(1, D4), _const(D3, D4),
            _const(D4, NCH), _const(D4, NCH), _const(D4, NCH),
            _const(1, NCH), _const(G, 1), _const(G, 1), _const(NCH, NCLS),
        ],
        out_specs=pl.BlockSpec((G, NCLS), lambda i: (0, 0)),
        out_shape=jax.ShapeDtypeStruct((G, NCLS), jnp.float32),
        scratch_shapes=[
            pltpu.VMEM((G, D4), jnp.float32),
            pltpu.VMEM((G, 1), jnp.float32),
        ],
    )(z4, z4, y4, dis, batch3, b4, W4, w0, w1, w2, convb, gcol, bcol, Rm)


def kernel(x, edge_index, edge_attr, batch, W1, b1, W2, b2, W3, b3, W4, b4,
           conv_w, conv_b, gamma, beta):
    f32 = jnp.float32
    src = edge_index[0].astype(jnp.int32)
    dst = edge_index[1].astype(jnp.int32)
    w = edge_attr[:, 4].astype(f32)

    pad = EP - E
    srcp = jnp.concatenate([src, jnp.zeros((pad,), jnp.int32)])
    dstp = jnp.concatenate([dst, jnp.zeros((pad,), jnp.int32)])
    wp = jnp.concatenate([w, jnp.zeros((pad,), f32)])
    wexp = jnp.broadcast_to(wp[:, None], (EP, 16))
    src2 = srcp.reshape(ECH, K)
    dst2 = dstp.reshape(ECH, K)

    z16 = jnp.zeros((N, 16), f32)
    z32 = jnp.zeros((N, 32), f32)
    z64 = jnp.zeros((N, 64), f32)

    xpad = jnp.pad(x.astype(f32), ((0, 0), (0, D1 - IC)))
    W1p = jnp.pad(W1, ((0, D1 - IC), (0, 0)))

    wk = conv_w[:, :, :, 0]
    w0 = wk[:, :, 0].T
    w1 = wk[:, :, 1].T
    w2 = wk[:, :, 2].T
    convb = conv_b.reshape(1, NCH)
    gcol = jnp.concatenate([gamma] * BS).reshape(G, 1)
    bcol = jnp.concatenate([beta] * BS).reshape(G, 1)
    cidx = jnp.arange(NCH, dtype=jnp.int32)
    Rm = (cidx[:, None] % NCLS ==
          jnp.arange(NCLS, dtype=jnp.int32)[None, :]).astype(f32)
    batch3 = batch.astype(jnp.int32).reshape(16, 1, RPT)

    deg2 = _sc_scatter(16, False)(src2, dst2, wexp, z16, z16)
    dis, y1 = _tc_prep(deg2, xpad)

    zl1 = _sc_scatter(16, True)(src2, dst2, wp, y1, z16)
    y2 = _tc_mid(zl1, y1, dis, b1, W1p)
    zl2 = _sc_scatter(16, True)(src2, dst2, wp, y2, z16)
    y3 = _tc_mid(zl2, y2, dis, b2, W2)
    zl3 = _sc_scatter(32, True)(src2, dst2, wp, y3, z32)
    y4 = _tc_mid(zl3, y3, dis, b3, W3)
    zl4 = _sc_scatter(64, True)(src2, dst2, wp, y4, z64)

    out = _tc_head(zl4, y4, dis, batch3, b4.reshape(1, D4), W4,
                   w0, w1, w2, convb, gcol, bcol, Rm)
    return out.reshape(BS, T, NCLS)
